# Initial kernel scaffold; baseline (speedup 1.0000x reference)
import jax
import jax.numpy as jnp
from jax.experimental import pallas as pl


def kernel(all_inputs, tables):
    raise NotImplementedError("write your pallas kernel here")


# speedup vs baseline: 18.8371x; 18.8371x over previous
import functools

import jax
import jax.numpy as jnp
from jax import lax
from jax.experimental import pallas as pl
from jax.experimental.pallas import tpu as pltpu
from jax.experimental.pallas import tpu_sc as plsc

NUM_FIELDS = 26
VOCAB = 100000
DIM = 32
B = 4096

_INFO = plsc.get_sparse_core_info()
_NC = _INFO.num_cores
_NS = _INFO.num_subcores
_NW = _NC * _NS
_N = B * NUM_FIELDS
_NB = _N // _NW
_LANES = 16


def _make_gather():
    mesh = plsc.VectorSubcoreMesh(core_axis_name="c", subcore_axis_name="s")

    @functools.partial(
        pl.kernel,
        mesh=mesh,
        out_type=jax.ShapeDtypeStruct((_N, DIM), jnp.float32),
        compiler_params=pltpu.CompilerParams(use_tc_tiling_on_sc=False),
        scratch_types=[
            pltpu.VMEM((_NB,), jnp.int32),
            pltpu.VMEM((_NB, DIM), jnp.float32),
            pltpu.SemaphoreType.DMA,
        ],
    )
    def gather_kernel(table_hbm, idx_hbm, out_hbm, idx_v, rows_v, sem):
        wid = lax.axis_index("s") * _NC + lax.axis_index("c")
        base = pl.multiple_of(wid * _NB, 8)
        pltpu.sync_copy(idx_hbm.at[pl.ds(base, _NB)], idx_v)

        lane = lax.iota(jnp.int32, _LANES)

        def fix(c, _):
            off = c * _LANES
            v = idx_v[pl.ds(off, _LANES)]
            field = lax.rem(base + off + lane, NUM_FIELDS)
            idx_v[pl.ds(off, _LANES)] = v + field * VOCAB
            return _

        lax.fori_loop(0, _NB // _LANES, fix, None)

        pltpu.async_copy(table_hbm.at[idx_v], rows_v, sem).wait()
        pltpu.sync_copy(rows_v, out_hbm.at[pl.ds(base, _NB)])

    return gather_kernel


_gather = _make_gather()


def kernel(all_inputs, tables):
    idx0 = all_inputs[:, 0, :].reshape(_N)
    table_flat = tables.reshape(NUM_FIELDS * VOCAB, DIM)
    out = _gather(table_flat, idx0)
    return out.reshape(B, NUM_FIELDS, DIM)


# speedup vs baseline: 110.4755x; 5.8648x over previous
import functools

import jax
import jax.numpy as jnp
from jax import lax
from jax.experimental import pallas as pl
from jax.experimental.pallas import tpu as pltpu
from jax.experimental.pallas import tpu_sc as plsc

NUM_FIELDS = 26
VOCAB = 100000
DIM = 32
B = 4096

_LANES = 16


def _make_gather():
    mesh = plsc.VectorSubcoreMesh(core_axis_name="c", subcore_axis_name="s")

    @functools.partial(
        pl.kernel,
        mesh=mesh,
        out_type=jax.ShapeDtypeStruct((NUM_FIELDS, DIM, B), jnp.float32),
        compiler_params=pltpu.CompilerParams(needs_layout_passes=False),
        scratch_types=[
            pltpu.VMEM((VOCAB,), jnp.float32),
            pltpu.VMEM((B,), jnp.int32),
            pltpu.VMEM((B,), jnp.float32),
        ],
    )
    def gather_kernel(table_hbm, idx_hbm, out_hbm, row_v, idx_v, out_v):
        d = lax.axis_index("s") * 2 + lax.axis_index("c")

        def per_field(f, _):
            pltpu.sync_copy(idx_hbm.at[f], idx_v)
            pltpu.sync_copy(table_hbm.at[f, d], row_v)

            def gath(c, _):
                off = c * _LANES
                vi = idx_v[pl.ds(off, _LANES)]
                out_v[pl.ds(off, _LANES)] = plsc.load_gather(row_v, [vi])
                return _

            lax.fori_loop(0, B // _LANES, gath, None)
            pltpu.sync_copy(out_v, out_hbm.at[f, d])
            return _

        lax.fori_loop(0, NUM_FIELDS, per_field, None)

    return gather_kernel


_gather = _make_gather()


def kernel(all_inputs, tables):
    idx_t = all_inputs[:, 0, :].T
    tables_t = tables.transpose(0, 2, 1)
    out_t = _gather(tables_t, idx_t)
    return out_t.transpose(2, 0, 1)
